# baseline (speedup 1.0000x reference)
import functools

import jax
import jax.numpy as jnp
from jax import lax
from jax.experimental import pallas as pl
from jax.experimental.pallas import tpu as pltpu

_TOP_RATIO = 0.3
_TOP_WEIGHT = 1.0
_IGNORE_INDEX = -1

_VB = 2000
_CH = 500
_NW = 1


def _lse_one(gt_row, pred_ref, lsep_ref, gathp_ref, block_idx):
    vb, n = pred_ref.shape
    nch = vb // _CH

    m = jnp.max(pred_ref[pl.ds(0, _CH), :], axis=0, keepdims=True)
    for ch in range(1, nch):
        x = pred_ref[pl.ds(ch * _CH, _CH), :]
        m = jnp.maximum(m, jnp.max(x, axis=0, keepdims=True))

    target = gt_row - block_idx * vb
    s = jnp.zeros((1, n), jnp.float32)
    g = jnp.zeros((1, n), jnp.float32)
    for ch in range(nch):
        x = pred_ref[pl.ds(ch * _CH, _CH), :]
        s = s + jnp.sum(jnp.exp(x - m), axis=0, keepdims=True)
        rows = lax.broadcasted_iota(jnp.int32, (_CH, n), 0) + ch * _CH
        g = g + jnp.sum(jnp.where(rows == target, x, jnp.float32(0.0)),
                        axis=0, keepdims=True)

    lsep_ref[...] = (m + jnp.log(s))[None]
    gathp_ref[...] = g[None]


def _lse_body(nb, gt_ref, *refs):
    j = pl.program_id(0)
    gt_row = gt_ref[...]
    preds = refs[:_NW]
    outs = refs[_NW:]
    for w in range(_NW):
        _lse_one(gt_row, preds[w], outs[2 * w], outs[2 * w + 1], w * nb + j)


def _lse_parts(pred_t, gt_row):
    c, n = pred_t.shape
    nb = c // (_VB * _NW)
    in_specs = [pl.BlockSpec((1, n), lambda j: (0, 0))] + [
        pl.BlockSpec((_VB, n), functools.partial(lambda w, j: (nb * w + j, 0), w))
        for w in range(_NW)
    ]
    out_specs = [pl.BlockSpec((1, 1, n), lambda j: (j, 0, 0))] * (2 * _NW)
    outs = pl.pallas_call(
        functools.partial(_lse_body, nb),
        grid=(nb,),
        in_specs=in_specs,
        out_specs=out_specs,
        out_shape=[jax.ShapeDtypeStruct((nb, 1, n), jnp.float32)] * (2 * _NW),
    )(gt_row, *([pred_t] * _NW))
    lsep = jnp.concatenate(outs[0::2], axis=0)
    gathp = jnp.concatenate(outs[1::2], axis=0)
    return lsep, gathp


def _loss_body(lsep_ref, gathp_ref, gt_ref, loss_ref, skey_ref):
    lsep = lsep_ref[...]
    m = jnp.max(lsep, axis=0, keepdims=True)
    s = jnp.sum(jnp.exp(lsep - m), axis=0, keepdims=True)
    lse = m + jnp.log(s)
    gat = jnp.sum(gathp_ref[...], axis=0, keepdims=True)

    nll = lse - gat
    valid = gt_ref[...] != _IGNORE_INDEX
    loss = jnp.where(valid, nll, jnp.float32(0.0))
    loss_ref[...] = loss.reshape(loss_ref.shape)
    bb = lax.bitcast_convert_type(loss, jnp.int32)
    skey = bb ^ (lax.shift_right_arithmetic(bb, 31) & jnp.int32(0x7FFFFFFF))
    skey_ref[...] = skey.reshape(skey_ref.shape)


def _loss_row(lsep, gathp, gt_row, n):
    return pl.pallas_call(
        _loss_body,
        out_shape=[
            jax.ShapeDtypeStruct((n,), jnp.float32),
            jax.ShapeDtypeStruct((n,), jnp.int32),
        ],
    )(lsep, gathp, gt_row)


def _lane_rot(x, sh):
    idx = ((lax.iota(jnp.int32, 16) + sh) & 15).reshape(16, 1)
    return lax.gather(
        x, idx,
        lax.GatherDimensionNumbers(
            offset_dims=(), collapsed_slice_dims=(0,), start_index_map=(0,)),
        slice_sizes=(1,),
        mode=lax.GatherScatterMode.PROMISE_IN_BOUNDS,
    )


def _lane_fold(x, op):
    for sh in (8, 4, 2, 1):
        x = op(x, _lane_rot(x, sh))
    return x


def _sc_topk(loss, skey, n, k):
    from jax.experimental.pallas import tpu_sc as plsc

    lanes = 16
    ng = n // lanes
    mesh = plsc.VectorSubcoreMesh(core_axis_name="c", subcore_axis_name="s")

    @functools.partial(
        pl.kernel,
        mesh=mesh,
        out_type=jax.ShapeDtypeStruct((lanes,), jnp.float32),
        scratch_types=[
            pltpu.VMEM((n,), jnp.float32),
            pltpu.VMEM((n,), jnp.int32),
        ],
    )
    def topk_k(loss_hbm, skey_hbm, out_hbm, loss_v, skey_v):
        wid = lax.axis_index("s") * 2 + lax.axis_index("c")

        @pl.when(wid == 0)
        def _():
            pltpu.sync_copy(loss_hbm, loss_v)
            pltpu.sync_copy(skey_hbm, skey_v)
            int_min = jnp.full((lanes,), -2147483648, jnp.int32)
            kvec = jnp.full((lanes,), k, jnp.int32)
            ione = jnp.full((lanes,), 1, jnp.int32)
            izero = jnp.zeros((lanes,), jnp.int32)

            unroll = 8
            nit = ng // unroll

            def tot_step(i, tot):
                base = i * (lanes * unroll)
                for u in range(unroll):
                    tot = tot + loss_v[pl.ds(base + u * lanes, lanes)]
                return tot

            tot = lax.fori_loop(0, nit, tot_step, jnp.zeros((lanes,), jnp.float32))
            total = _lane_fold(tot, jnp.add)

            def bit_step(t, p):
                one = jnp.full((lanes,), 1, jnp.int32)
                shift = jnp.full((lanes,), 31, jnp.int32) - t
                cand = p | lax.shift_left(one, shift)
                thresh = cand ^ int_min

                def cnt_step(i, cnt):
                    base = i * (lanes * unroll)
                    for u in range(unroll):
                        sk = skey_v[pl.ds(base + u * lanes, lanes)]
                        cnt = cnt + jnp.where(sk >= thresh, ione, izero)
                    return cnt

                cnt = lax.fori_loop(0, nit, cnt_step,
                                    jnp.zeros((lanes,), jnp.int32))
                cnt = _lane_fold(cnt, jnp.add)
                return jnp.where(cnt >= kvec, cand, p)

            p = lax.fori_loop(0, 32, bit_step, jnp.zeros((lanes,), jnp.int32))
            skey_th = p ^ int_min

            def tail_step(i, carry):
                cnt_gt, sum_gt, f_th_acc = carry
                base = i * (lanes * unroll)
                for u in range(unroll):
                    sk = skey_v[pl.ds(base + u * lanes, lanes)]
                    v = loss_v[pl.ds(base + u * lanes, lanes)]
                    over = sk > skey_th
                    cnt_gt = cnt_gt + jnp.where(over, ione, izero)
                    sum_gt = sum_gt + jnp.where(over, v, jnp.float32(0.0))
                    f_th_acc = jnp.maximum(
                        f_th_acc, jnp.where(sk == skey_th, v, -jnp.inf))
                return cnt_gt, sum_gt, f_th_acc

            cnt_gt, sum_gt, f_th_acc = lax.fori_loop(
                0, nit, tail_step,
                (jnp.zeros((lanes,), jnp.int32),
                 jnp.zeros((lanes,), jnp.float32),
                 jnp.full((lanes,), -jnp.inf, jnp.float32)))
            f_th = _lane_fold(f_th_acc, jnp.maximum)
            cnt_gt = _lane_fold(cnt_gt, jnp.add)
            topk_sum = (_lane_fold(sum_gt, jnp.add)
                        + (kvec - cnt_gt).astype(jnp.float32) * f_th)

            out = (total / jnp.float32(n)
                   + jnp.float32(_TOP_WEIGHT) * topk_sum / jnp.float32(k))
            loss_v[pl.ds(0, lanes)] = out
            pltpu.sync_copy(loss_v.at[pl.ds(0, lanes)], out_hbm)

    return topk_k(loss, skey)


def kernel(pred, gt):
    n, c = pred.shape
    k = max(int(_TOP_RATIO * n), 1)
    gt_row = gt.reshape(1, n)
    lsep, gathp = _lse_parts(pred.T, gt_row)
    nb = c // _VB
    loss, skey = _loss_row(lsep.reshape(nb, n), gathp.reshape(nb, n), gt_row, n)
    out = _sc_topk(loss, skey, n, k)
    return out[0]


# speedup vs baseline: 1.0133x; 1.0133x over previous
import functools

import jax
import jax.numpy as jnp
from jax import lax
from jax.experimental import pallas as pl
from jax.experimental.pallas import tpu as pltpu

_TOP_RATIO = 0.3
_TOP_WEIGHT = 1.0
_IGNORE_INDEX = -1

_VB = 2000
_CH = 500
_NW = 1


def _lse_body(nb, gt_ref, pred_ref, loss_ref, skey_ref, m_s, s_s, g_s):
    j = pl.program_id(0)
    gt_row = gt_ref[...]
    vb, n = pred_ref.shape
    nch = vb // _CH

    @pl.when(j == 0)
    def _():
        m_s[...] = jnp.full((1, n), -jnp.inf, jnp.float32)
        s_s[...] = jnp.zeros((1, n), jnp.float32)
        g_s[...] = jnp.zeros((1, n), jnp.float32)

    bm = jnp.max(pred_ref[pl.ds(0, _CH), :], axis=0, keepdims=True)
    for ch in range(1, nch):
        x = pred_ref[pl.ds(ch * _CH, _CH), :]
        bm = jnp.maximum(bm, jnp.max(x, axis=0, keepdims=True))

    target = gt_row - j * vb
    bs = jnp.zeros((1, n), jnp.float32)
    bg = jnp.zeros((1, n), jnp.float32)
    for ch in range(nch):
        x = pred_ref[pl.ds(ch * _CH, _CH), :]
        bs = bs + jnp.sum(jnp.exp(x - bm), axis=0, keepdims=True)
        rows = lax.broadcasted_iota(jnp.int32, (_CH, n), 0) + ch * _CH
        bg = bg + jnp.sum(jnp.where(rows == target, x, jnp.float32(0.0)),
                          axis=0, keepdims=True)

    m_old = m_s[...]
    m_new = jnp.maximum(m_old, bm)
    s_new = s_s[...] * jnp.exp(m_old - m_new) + bs * jnp.exp(bm - m_new)
    g_new = g_s[...] + bg
    m_s[...] = m_new
    s_s[...] = s_new
    g_s[...] = g_new

    @pl.when(j == nb - 1)
    def _():
        lse = m_new + jnp.log(s_new)
        nll = lse - g_new
        valid = gt_row != _IGNORE_INDEX
        loss = jnp.where(valid, nll, jnp.float32(0.0))
        loss_ref[...] = loss.reshape(loss_ref.shape)
        bb = lax.bitcast_convert_type(loss, jnp.int32)
        skey = bb ^ (lax.shift_right_arithmetic(bb, 31) & jnp.int32(0x7FFFFFFF))
        skey_ref[...] = skey.reshape(skey_ref.shape)


def _loss_row(pred_t, gt_row):
    c, n = pred_t.shape
    nb = c // _VB
    return pl.pallas_call(
        functools.partial(_lse_body, nb),
        grid=(nb,),
        in_specs=[
            pl.BlockSpec((1, n), lambda j: (0, 0)),
            pl.BlockSpec((_VB, n), lambda j: (j, 0)),
        ],
        out_specs=[
            pl.BlockSpec((n,), lambda j: (0,)),
            pl.BlockSpec((n,), lambda j: (0,)),
        ],
        out_shape=[
            jax.ShapeDtypeStruct((n,), jnp.float32),
            jax.ShapeDtypeStruct((n,), jnp.int32),
        ],
        scratch_shapes=[
            pltpu.VMEM((1, n), jnp.float32),
            pltpu.VMEM((1, n), jnp.float32),
            pltpu.VMEM((1, n), jnp.float32),
        ],
    )(gt_row, pred_t)


def _lane_rot(x, sh):
    idx = ((lax.iota(jnp.int32, 16) + sh) & 15).reshape(16, 1)
    return lax.gather(
        x, idx,
        lax.GatherDimensionNumbers(
            offset_dims=(), collapsed_slice_dims=(0,), start_index_map=(0,)),
        slice_sizes=(1,),
        mode=lax.GatherScatterMode.PROMISE_IN_BOUNDS,
    )


def _lane_fold(x, op):
    for sh in (8, 4, 2, 1):
        x = op(x, _lane_rot(x, sh))
    return x


def _sc_topk(loss, skey, n, k):
    from jax.experimental.pallas import tpu_sc as plsc

    lanes = 16
    ng = n // lanes
    mesh = plsc.VectorSubcoreMesh(core_axis_name="c", subcore_axis_name="s")

    @functools.partial(
        pl.kernel,
        mesh=mesh,
        out_type=jax.ShapeDtypeStruct((lanes,), jnp.float32),
        scratch_types=[
            pltpu.VMEM((n,), jnp.float32),
            pltpu.VMEM((n,), jnp.int32),
        ],
    )
    def topk_k(loss_hbm, skey_hbm, out_hbm, loss_v, skey_v):
        wid = lax.axis_index("s") * 2 + lax.axis_index("c")

        @pl.when(wid == 0)
        def _():
            pltpu.sync_copy(loss_hbm, loss_v)
            pltpu.sync_copy(skey_hbm, skey_v)
            int_min = jnp.full((lanes,), -2147483648, jnp.int32)
            kvec = jnp.full((lanes,), k, jnp.int32)
            ione = jnp.full((lanes,), 1, jnp.int32)
            izero = jnp.zeros((lanes,), jnp.int32)

            unroll = 8
            nit = ng // unroll

            def tot_step(i, tot):
                base = i * (lanes * unroll)
                for u in range(unroll):
                    tot = tot + loss_v[pl.ds(base + u * lanes, lanes)]
                return tot

            tot = lax.fori_loop(0, nit, tot_step, jnp.zeros((lanes,), jnp.float32))
            total = _lane_fold(tot, jnp.add)

            def bit_step(t, p):
                one = jnp.full((lanes,), 1, jnp.int32)
                shift = jnp.full((lanes,), 31, jnp.int32) - t
                cand = p | lax.shift_left(one, shift)
                thresh = cand ^ int_min

                def cnt_step(i, cnt):
                    base = i * (lanes * unroll)
                    for u in range(unroll):
                        sk = skey_v[pl.ds(base + u * lanes, lanes)]
                        cnt = cnt + jnp.where(sk >= thresh, ione, izero)
                    return cnt

                cnt = lax.fori_loop(0, nit, cnt_step,
                                    jnp.zeros((lanes,), jnp.int32))
                cnt = _lane_fold(cnt, jnp.add)
                return jnp.where(cnt >= kvec, cand, p)

            p = lax.fori_loop(0, 32, bit_step, jnp.zeros((lanes,), jnp.int32))
            skey_th = p ^ int_min

            def tail_step(i, carry):
                cnt_gt, sum_gt, f_th_acc = carry
                base = i * (lanes * unroll)
                for u in range(unroll):
                    sk = skey_v[pl.ds(base + u * lanes, lanes)]
                    v = loss_v[pl.ds(base + u * lanes, lanes)]
                    over = sk > skey_th
                    cnt_gt = cnt_gt + jnp.where(over, ione, izero)
                    sum_gt = sum_gt + jnp.where(over, v, jnp.float32(0.0))
                    f_th_acc = jnp.maximum(
                        f_th_acc, jnp.where(sk == skey_th, v, -jnp.inf))
                return cnt_gt, sum_gt, f_th_acc

            cnt_gt, sum_gt, f_th_acc = lax.fori_loop(
                0, nit, tail_step,
                (jnp.zeros((lanes,), jnp.int32),
                 jnp.zeros((lanes,), jnp.float32),
                 jnp.full((lanes,), -jnp.inf, jnp.float32)))
            f_th = _lane_fold(f_th_acc, jnp.maximum)
            cnt_gt = _lane_fold(cnt_gt, jnp.add)
            topk_sum = (_lane_fold(sum_gt, jnp.add)
                        + (kvec - cnt_gt).astype(jnp.float32) * f_th)

            out = (total / jnp.float32(n)
                   + jnp.float32(_TOP_WEIGHT) * topk_sum / jnp.float32(k))
            loss_v[pl.ds(0, lanes)] = out
            pltpu.sync_copy(loss_v.at[pl.ds(0, lanes)], out_hbm)

    return topk_k(loss, skey)


def kernel(pred, gt):
    n, c = pred.shape
    k = max(int(_TOP_RATIO * n), 1)
    gt_row = gt.reshape(1, n)
    loss, skey = _loss_row(pred.T, gt_row)
    out = _sc_topk(loss, skey, n, k)
    return out[0]
